# baseline (speedup 1.0000x reference)
import functools

import jax
import jax.numpy as jnp
from jax.experimental import pallas as pl
from jax.experimental.pallas import tpu as pltpu

_C = 19
_BH = 128


def _body(pp_ref, d_ref, lab_ref, s_out, c_out, s_rows, c_rows):
    h = pl.program_id(1)
    nh = pl.num_programs(1)

    @pl.when(h == 0)
    def _init():
        s_rows[...] = jnp.zeros_like(s_rows)
        c_rows[...] = jnp.zeros_like(c_rows)

    x = pp_ref[0]
    lab = lab_ref[0]

    m = x[0]
    for c in range(1, _C):
        m = jnp.maximum(m, x[c])
    ssum = jnp.exp(x[0] - m)
    xl = jnp.where(lab == 0, x[0], 0.0)
    for c in range(1, _C):
        ssum = ssum + jnp.exp(x[c] - m)
        xl = jnp.where(lab == c, x[c], xl)
    lse = m + jnp.log(ssum)

    dp = d_ref[0, 0]
    fool = jnp.maximum(dp, 0.0) + jnp.log1p(jnp.exp(-jnp.abs(dp)))
    p = fool * (lse - xl)

    for c in range(_C):
        mask = lab == c
        s_rows[c : c + 1, :] = s_rows[c : c + 1, :] + jnp.sum(
            jnp.where(mask, p, 0.0), axis=0, keepdims=True
        )
        c_rows[c : c + 1, :] = c_rows[c : c + 1, :] + jnp.sum(
            jnp.where(mask, 1.0, 0.0), axis=0, keepdims=True
        )

    @pl.when(h == nh - 1)
    def _finalize():
        lane = jax.lax.broadcasted_iota(jnp.int32, (1, 128), 1)
        sv = jnp.zeros((1, 128), jnp.float32)
        cv = jnp.zeros((1, 128), jnp.float32)
        for c in range(_C):
            oh = jnp.where(lane == c, 1.0, 0.0)
            sv = sv + jnp.sum(s_rows[c : c + 1, :], axis=1, keepdims=True) * oh
            cv = cv + jnp.sum(c_rows[c : c + 1, :], axis=1, keepdims=True) * oh
        s_out[0] = sv
        c_out[0] = cv


@functools.partial(jax.jit, static_argnames=("interpret",))
def kernel(parser_prediction, discriminator_pred, labels, interpret=False):
    b, c, hh, w = parser_prediction.shape
    labels = labels.astype(jnp.int32)
    nh = hh // _BH
    s_out, c_out = pl.pallas_call(
        _body,
        grid=(b, nh),
        in_specs=[
            pl.BlockSpec((1, c, _BH, w), lambda i, j: (i, 0, j, 0)),
            pl.BlockSpec((1, 1, _BH, w), lambda i, j: (i, 0, j, 0)),
            pl.BlockSpec((1, _BH, w), lambda i, j: (i, j, 0)),
        ],
        out_specs=[
            pl.BlockSpec((1, 1, 128), lambda i, j: (i, 0, 0)),
            pl.BlockSpec((1, 1, 128), lambda i, j: (i, 0, 0)),
        ],
        out_shape=[
            jax.ShapeDtypeStruct((b, 1, 128), jnp.float32),
            jax.ShapeDtypeStruct((b, 1, 128), jnp.float32),
        ],
        scratch_shapes=[
            pltpu.VMEM((_C, w), jnp.float32),
            pltpu.VMEM((_C, w), jnp.float32),
        ],
        compiler_params=pltpu.CompilerParams(
            dimension_semantics=("parallel", "arbitrary"),
        ),
        interpret=interpret,
    )(parser_prediction, discriminator_pred, labels)
    s = s_out[:, 0, :_C]
    cnt = c_out[:, 0, :_C]
    tot = jnp.float32(hh * w)
    return jnp.sum(s * (1.0 - cnt / tot)) / (b * tot)


# speedup vs baseline: 1.0886x; 1.0886x over previous
import functools

import jax
import jax.numpy as jnp
from jax.experimental import pallas as pl
from jax.experimental.pallas import tpu as pltpu

_C = 19
_BH = 32


def _body(pp_ref, d_ref, lab_ref, s_out, c_out, s_rows, c_rows):
    h = pl.program_id(1)
    nh = pl.num_programs(1)

    @pl.when(h == 0)
    def _init():
        s_rows[...] = jnp.zeros_like(s_rows)
        c_rows[...] = jnp.zeros_like(c_rows)

    x = pp_ref[0]
    lab = lab_ref[0]

    m = x[0]
    for c in range(1, _C):
        m = jnp.maximum(m, x[c])
    ssum = jnp.exp(x[0] - m)
    xl = jnp.where(lab == 0, x[0], 0.0)
    for c in range(1, _C):
        ssum = ssum + jnp.exp(x[c] - m)
        xl = jnp.where(lab == c, x[c], xl)
    lse = m + jnp.log(ssum)

    dp = d_ref[0, 0]
    fool = jnp.maximum(dp, 0.0) + jnp.log1p(jnp.exp(-jnp.abs(dp)))
    p = fool * (lse - xl)

    for c in range(_C):
        mask = lab == c
        s_rows[c : c + 1, :] = s_rows[c : c + 1, :] + jnp.sum(
            jnp.where(mask, p, 0.0), axis=0, keepdims=True
        )
        c_rows[c : c + 1, :] = c_rows[c : c + 1, :] + jnp.sum(
            jnp.where(mask, 1.0, 0.0), axis=0, keepdims=True
        )

    @pl.when(h == nh - 1)
    def _finalize():
        lane = jax.lax.broadcasted_iota(jnp.int32, (1, 128), 1)
        sv = jnp.zeros((1, 128), jnp.float32)
        cv = jnp.zeros((1, 128), jnp.float32)
        for c in range(_C):
            oh = jnp.where(lane == c, 1.0, 0.0)
            sv = sv + jnp.sum(s_rows[c : c + 1, :], axis=1, keepdims=True) * oh
            cv = cv + jnp.sum(c_rows[c : c + 1, :], axis=1, keepdims=True) * oh
        s_out[0] = sv
        c_out[0] = cv


@functools.partial(jax.jit, static_argnames=("interpret",))
def kernel(parser_prediction, discriminator_pred, labels, interpret=False):
    b, c, hh, w = parser_prediction.shape
    labels = labels.astype(jnp.int32)
    nh = hh // _BH
    s_out, c_out = pl.pallas_call(
        _body,
        grid=(b, nh),
        in_specs=[
            pl.BlockSpec((1, c, _BH, w), lambda i, j: (i, 0, j, 0)),
            pl.BlockSpec((1, 1, _BH, w), lambda i, j: (i, 0, j, 0)),
            pl.BlockSpec((1, _BH, w), lambda i, j: (i, j, 0)),
        ],
        out_specs=[
            pl.BlockSpec((1, 1, 128), lambda i, j: (i, 0, 0)),
            pl.BlockSpec((1, 1, 128), lambda i, j: (i, 0, 0)),
        ],
        out_shape=[
            jax.ShapeDtypeStruct((b, 1, 128), jnp.float32),
            jax.ShapeDtypeStruct((b, 1, 128), jnp.float32),
        ],
        scratch_shapes=[
            pltpu.VMEM((_C, w), jnp.float32),
            pltpu.VMEM((_C, w), jnp.float32),
        ],
        compiler_params=pltpu.CompilerParams(
            dimension_semantics=("parallel", "arbitrary"),
        ),
        interpret=interpret,
    )(parser_prediction, discriminator_pred, labels)
    s = s_out[:, 0, :_C]
    cnt = c_out[:, 0, :_C]
    tot = jnp.float32(hh * w)
    return jnp.sum(s * (1.0 - cnt / tot)) / (b * tot)
